# baseline (speedup 1.0000x reference)
import functools

import jax
import jax.numpy as jnp
from jax import lax
from jax.experimental import pallas as pl
from jax.experimental.pallas import tpu as pltpu
from jax.experimental.pallas import tpu_sc as plsc

N_VOCAB = 1000000
D = 300
DPAD = 384
B = 1024
L = 50
HEAD = 256
TOFF = 256

NC = 2
NS = 16
NW = NC * NS
B_PER_W = B // NW
NBUF = 2

TR_BLK = 2048


def _tr_body(tt_ref, out_ref):
    out_ref[...] = tt_ref[...].T


_transpose = pl.pallas_call(
    _tr_body,
    grid=(pl.cdiv(N_VOCAB, TR_BLK),),
    in_specs=[pl.BlockSpec((D, TR_BLK), lambda i: (0, i))],
    out_specs=pl.BlockSpec((TR_BLK, D), lambda i: (i, 0)),
    out_shape=jax.ShapeDtypeStruct((N_VOCAB, D), jnp.float32),
)


@functools.partial(
    pl.kernel,
    out_type=jax.ShapeDtypeStruct((B, L, D), jnp.float32),
    mesh=plsc.VectorSubcoreMesh(core_axis_name="c", subcore_axis_name="s"),
    scratch_types=[
        pltpu.VMEM((B_PER_W, L), jnp.int32),
        pltpu.VMEM((NBUF, L, 128), jnp.float32),
        pltpu.VMEM((NBUF, L, 128), jnp.float32),
        pltpu.VMEM((NBUF, L, 128), jnp.float32),
        pltpu.SemaphoreType.DMA,
        pltpu.SemaphoreType.DMA,
        pltpu.SemaphoreType.DMA,
        pltpu.SemaphoreType.DMA,
    ],
)
def _sc_gather(idx_hbm, table_hbm, out_hbm, idx_v, buf0_v, buf1_v, buf2_v,
               gsem0, gsem1, wsem0, wsem1):
    wid = lax.axis_index("s") * NC + lax.axis_index("c")
    toff = pl.multiple_of(wid * 0 + TOFF, 128)
    gsems = (gsem0, gsem1)
    wsems = (wsem0, wsem1)
    bufs = (buf0_v, buf1_v, buf2_v)
    pltpu.sync_copy(idx_hbm.at[wid], idx_v)

    def g_refs(k, buf):
        return (
            (table_hbm.at[idx_v.at[k], pl.ds(0, 128)], bufs[0].at[buf]),
            (table_hbm.at[idx_v.at[k], pl.ds(128, 128)], bufs[1].at[buf]),
            (table_hbm.at[idx_v.at[k], pl.ds(toff, 128)], bufs[2].at[buf]),
        )

    def w_refs(k, buf):
        bb = wid * B_PER_W + k
        return (
            (buf0_v.at[buf], out_hbm.at[bb, :, pl.ds(0, 128)]),
            (buf1_v.at[buf], out_hbm.at[bb, :, pl.ds(128, 128)]),
            (buf2_v.at[buf], out_hbm.at[bb, :, pl.ds(toff, 128)]),
        )

    def issue(refs, sem):
        for src, dst in refs:
            pltpu.async_copy(src, dst, sem)

    def drain(refs, sem):
        for src, dst in refs:
            pltpu.make_async_copy(src, dst, sem).wait()

    for buf in range(NBUF):
        issue(g_refs(buf, buf), gsems[buf])

    def body(k, _):
        for buf in range(NBUF):
            kk = k * NBUF + buf
            drain(g_refs(kk, buf), gsems[buf])
            issue(w_refs(kk, buf), wsems[buf])

            @pl.when(kk + NBUF < B_PER_W)
            def _():
                drain(w_refs(kk, buf), wsems[buf])
                issue(g_refs(kk + NBUF, buf), gsems[buf])
        return ()

    lax.fori_loop(0, B_PER_W // NBUF, body, (), unroll=False)
    for buf in range(NBUF):
        k = B_PER_W - NBUF + buf
        drain(w_refs(k, buf), wsems[buf])


def kernel(voc, voc_emb_weight):
    table = _transpose(voc_emb_weight.T)
    idx = voc[:, 0, :].reshape(NW, B_PER_W, L)
    return _sc_gather(idx, table)


# speedup vs baseline: 1.0740x; 1.0740x over previous
import functools

import jax
import jax.numpy as jnp
from jax import lax
from jax.experimental import pallas as pl
from jax.experimental.pallas import tpu as pltpu
from jax.experimental.pallas import tpu_sc as plsc

N_VOCAB = 1000000
D = 300
DPAD = 384
B = 1024
L = 50
HEAD = 256
TOFF = 256

NC = 2
NS = 16
NW = NC * NS
B_PER_W = B // NW
NBUF = 2

TR_BLK = 4096


def _tr_body(tt_ref, out_ref):
    out_ref[...] = tt_ref[...].T


_transpose = pl.pallas_call(
    _tr_body,
    grid=(pl.cdiv(N_VOCAB, TR_BLK),),
    in_specs=[pl.BlockSpec((D, TR_BLK), lambda i: (0, i))],
    out_specs=pl.BlockSpec((TR_BLK, D), lambda i: (i, 0)),
    out_shape=jax.ShapeDtypeStruct((N_VOCAB, D), jnp.float32),
)


@functools.partial(
    pl.kernel,
    out_type=jax.ShapeDtypeStruct((B, L, D), jnp.float32),
    mesh=plsc.VectorSubcoreMesh(core_axis_name="c", subcore_axis_name="s"),
    scratch_types=[
        pltpu.VMEM((B_PER_W, L), jnp.int32),
        pltpu.VMEM((NBUF, L, 128), jnp.float32),
        pltpu.VMEM((NBUF, L, 128), jnp.float32),
        pltpu.VMEM((NBUF, L, 128), jnp.float32),
        pltpu.SemaphoreType.DMA,
        pltpu.SemaphoreType.DMA,
        pltpu.SemaphoreType.DMA,
        pltpu.SemaphoreType.DMA,
    ],
)
def _sc_gather(idx_hbm, table_hbm, out_hbm, idx_v, buf0_v, buf1_v, buf2_v,
               gsem0, gsem1, wsem0, wsem1):
    wid = lax.axis_index("s") * NC + lax.axis_index("c")
    toff = pl.multiple_of(wid * 0 + TOFF, 128)
    gsems = (gsem0, gsem1)
    wsems = (wsem0, wsem1)
    bufs = (buf0_v, buf1_v, buf2_v)
    pltpu.sync_copy(idx_hbm.at[wid], idx_v)

    def g_refs(k, buf):
        return (
            (table_hbm.at[idx_v.at[k], pl.ds(0, 128)], bufs[0].at[buf]),
            (table_hbm.at[idx_v.at[k], pl.ds(128, 128)], bufs[1].at[buf]),
            (table_hbm.at[idx_v.at[k], pl.ds(toff, 128)], bufs[2].at[buf]),
        )

    def w_refs(k, buf):
        bb = wid * B_PER_W + k
        return (
            (buf0_v.at[buf], out_hbm.at[bb, :, pl.ds(0, 128)]),
            (buf1_v.at[buf], out_hbm.at[bb, :, pl.ds(128, 128)]),
            (buf2_v.at[buf], out_hbm.at[bb, :, pl.ds(toff, 128)]),
        )

    def issue(refs, sem):
        for src, dst in refs:
            pltpu.async_copy(src, dst, sem)

    def drain(refs, sem):
        for src, dst in refs:
            pltpu.make_async_copy(src, dst, sem).wait()

    for buf in range(NBUF):
        issue(g_refs(buf, buf), gsems[buf])

    def body(k, _):
        for buf in range(NBUF):
            kk = k * NBUF + buf
            drain(g_refs(kk, buf), gsems[buf])
            issue(w_refs(kk, buf), wsems[buf])

            @pl.when(kk + NBUF < B_PER_W)
            def _():
                drain(w_refs(kk, buf), wsems[buf])
                issue(g_refs(kk + NBUF, buf), gsems[buf])
        return ()

    lax.fori_loop(0, B_PER_W // NBUF, body, (), unroll=False)
    for buf in range(NBUF):
        k = B_PER_W - NBUF + buf
        drain(w_refs(k, buf), wsems[buf])


def kernel(voc, voc_emb_weight):
    table = _transpose(voc_emb_weight.T)
    idx = voc[:, 0, :].reshape(NW, B_PER_W, L)
    return _sc_gather(idx, table)


# speedup vs baseline: 1.0908x; 1.0157x over previous
import functools

import jax
import jax.numpy as jnp
from jax import lax
from jax.experimental import pallas as pl
from jax.experimental.pallas import tpu as pltpu
from jax.experimental.pallas import tpu_sc as plsc

N_VOCAB = 1000000
D = 300
DPAD = 384
B = 1024
L = 50
HEAD = 256
TOFF = 256

NC = 2
NS = 16
NW = NC * NS
B_PER_W = B // NW
NBUF = 2

TR_BLK = 8192


def _tr_body(tt_ref, out_ref):
    out_ref[...] = tt_ref[...].T


_transpose = pl.pallas_call(
    _tr_body,
    grid=(pl.cdiv(N_VOCAB, TR_BLK),),
    in_specs=[pl.BlockSpec((D, TR_BLK), lambda i: (0, i))],
    out_specs=pl.BlockSpec((TR_BLK, D), lambda i: (i, 0)),
    out_shape=jax.ShapeDtypeStruct((N_VOCAB, D), jnp.float32),
)


@functools.partial(
    pl.kernel,
    out_type=jax.ShapeDtypeStruct((B, L, D), jnp.float32),
    mesh=plsc.VectorSubcoreMesh(core_axis_name="c", subcore_axis_name="s"),
    scratch_types=[
        pltpu.VMEM((B_PER_W, L), jnp.int32),
        pltpu.VMEM((NBUF, L, 128), jnp.float32),
        pltpu.VMEM((NBUF, L, 128), jnp.float32),
        pltpu.VMEM((NBUF, L, 128), jnp.float32),
        pltpu.SemaphoreType.DMA,
        pltpu.SemaphoreType.DMA,
        pltpu.SemaphoreType.DMA,
        pltpu.SemaphoreType.DMA,
    ],
)
def _sc_gather(idx_hbm, table_hbm, out_hbm, idx_v, buf0_v, buf1_v, buf2_v,
               gsem0, gsem1, wsem0, wsem1):
    wid = lax.axis_index("s") * NC + lax.axis_index("c")
    toff = pl.multiple_of(wid * 0 + TOFF, 128)
    gsems = (gsem0, gsem1)
    wsems = (wsem0, wsem1)
    bufs = (buf0_v, buf1_v, buf2_v)
    pltpu.sync_copy(idx_hbm.at[wid], idx_v)

    def g_refs(k, buf):
        return (
            (table_hbm.at[idx_v.at[k], pl.ds(0, 128)], bufs[0].at[buf]),
            (table_hbm.at[idx_v.at[k], pl.ds(128, 128)], bufs[1].at[buf]),
            (table_hbm.at[idx_v.at[k], pl.ds(toff, 128)], bufs[2].at[buf]),
        )

    def w_refs(k, buf):
        bb = wid * B_PER_W + k
        return (
            (buf0_v.at[buf], out_hbm.at[bb, :, pl.ds(0, 128)]),
            (buf1_v.at[buf], out_hbm.at[bb, :, pl.ds(128, 128)]),
            (buf2_v.at[buf], out_hbm.at[bb, :, pl.ds(toff, 128)]),
        )

    def issue(refs, sem):
        for src, dst in refs:
            pltpu.async_copy(src, dst, sem)

    def drain(refs, sem):
        for src, dst in refs:
            pltpu.make_async_copy(src, dst, sem).wait()

    for buf in range(NBUF):
        issue(g_refs(buf, buf), gsems[buf])

    def body(k, _):
        for buf in range(NBUF):
            kk = k * NBUF + buf
            drain(g_refs(kk, buf), gsems[buf])
            issue(w_refs(kk, buf), wsems[buf])

            @pl.when(kk + NBUF < B_PER_W)
            def _():
                drain(w_refs(kk, buf), wsems[buf])
                issue(g_refs(kk + NBUF, buf), gsems[buf])
        return ()

    lax.fori_loop(0, B_PER_W // NBUF, body, (), unroll=False)
    for buf in range(NBUF):
        k = B_PER_W - NBUF + buf
        drain(w_refs(k, buf), wsems[buf])


def kernel(voc, voc_emb_weight):
    table = _transpose(voc_emb_weight.T)
    idx = voc[:, 0, :].reshape(NW, B_PER_W, L)
    return _sc_gather(idx, table)
